# baseline (speedup 1.0000x reference)
import functools

import jax
import jax.numpy as jnp
from jax import lax
from jax.experimental import pallas as pl
from jax.experimental.pallas import tpu as pltpu
from jax.experimental.pallas import tpu_sc as plsc

VOCAB = 1000
BATCH = 1024
SEQ_LEN = 50
TOKENS = BATCH * SEQ_LEN

NUM_CORES = 2
NUM_SUBCORES = 16
NW = NUM_CORES * NUM_SUBCORES
LANES = 16

NCB = 4
CB = BATCH // NCB
NVQ = 8
VN = 128
VLAST = VOCAB - VN
BGROUPS = CB // LANES
NBUF = 2

_mesh = plsc.VectorSubcoreMesh(core_axis_name="c", subcore_axis_name="s")


@functools.partial(
    pl.kernel,
    mesh=_mesh,
    out_type=jax.ShapeDtypeStruct((SEQ_LEN, VOCAB, BATCH), jnp.float32),
    scratch_types=[
        pltpu.VMEM((CB * SEQ_LEN,), jnp.int32),
    ]
    + [pltpu.VMEM((VN, CB), jnp.float32) for _ in range(NBUF)]
    + [pltpu.SemaphoreType.DMA] * NBUF,
    compiler_params=pltpu.CompilerParams(needs_layout_passes=False),
)
def _onehot_sc(idx_hbm, out_hbm, idx_v, *bufs_and_sems):
    bufs = bufs_and_sems[:NBUF]
    sems = bufs_and_sems[NBUF:]
    cid = lax.axis_index("c")
    sid = lax.axis_index("s")
    wid = sid * NUM_CORES + cid
    cb = wid % NCB
    vlo = jnp.minimum((wid // NCB) * VN, VLAST)

    pltpu.sync_copy(idx_hbm.at[pl.ds(cb * CB * SEQ_LEN, CB * SEQ_LEN)], idx_v)

    lanes = lax.iota(jnp.int32, 16)
    ones = jnp.ones((LANES,), jnp.float32)
    zeros = jnp.zeros((LANES,), jnp.float32)

    for buf in bufs:
        def zero_row(r, c, buf=buf):
            for j in range(CB // LANES):
                buf[r, pl.ds(j * LANES, LANES)] = zeros
            return c

        lax.fori_loop(0, VN, zero_row, 0)

    def scatter_slab(buf, s, vals):
        for g in range(BGROUPS):
            bloc = g * LANES + lanes
            v = plsc.load_gather(idx_v, [bloc * SEQ_LEN + s])
            vloc = v - vlo
            mask = (vloc >= 0) & (vloc < VN)
            plsc.store_scatter(buf, [vloc, bloc], vals, mask=mask)

    def dma(r, s):
        return pltpu.make_async_copy(
            bufs[r],
            out_hbm.at[s, pl.ds(vlo, VN), pl.ds(cb * CB, CB)],
            sems[r],
        )

    def pair_body(p, c):
        for r in range(NBUF):
            s = p * NBUF + r

            @pl.when(p > 0)
            def _():
                dma(r, s - NBUF).wait()
                scatter_slab(bufs[r], s - NBUF, zeros)

            scatter_slab(bufs[r], s, ones)
            dma(r, s).start()
        return c

    lax.fori_loop(0, SEQ_LEN // NBUF, pair_body, 0)

    for r in range(NBUF):
        dma(r, SEQ_LEN - NBUF + r).wait()


def kernel(indices, table):
    del table
    out_t = _onehot_sc(indices.reshape(TOKENS))
    return jnp.transpose(out_t, (2, 0, 1))


# speedup vs baseline: 1.0089x; 1.0089x over previous
import functools

import jax
import jax.numpy as jnp
from jax import lax
from jax.experimental import pallas as pl
from jax.experimental.pallas import tpu as pltpu
from jax.experimental.pallas import tpu_sc as plsc

VOCAB = 1000
BATCH = 1024
SEQ_LEN = 50
TOKENS = BATCH * SEQ_LEN

NUM_CORES = 2
NUM_SUBCORES = 16
NW = NUM_CORES * NUM_SUBCORES
LANES = 16

NCB = 8
CB = BATCH // NCB
NVQ = 4
VN = 256
VSTEP = (VOCAB - VN) // (NVQ - 1)
BGROUPS = CB // LANES
NBUF = 2

_mesh = plsc.VectorSubcoreMesh(core_axis_name="c", subcore_axis_name="s")


@functools.partial(
    pl.kernel,
    mesh=_mesh,
    out_type=jax.ShapeDtypeStruct((SEQ_LEN, VOCAB, BATCH), jnp.float32),
    scratch_types=[
        pltpu.VMEM((SEQ_LEN, CB), jnp.int32),
    ]
    + [pltpu.VMEM((VN, CB), jnp.float32) for _ in range(NBUF)]
    + [pltpu.SemaphoreType.DMA] * NBUF,
    compiler_params=pltpu.CompilerParams(needs_layout_passes=False),
)
def _onehot_sc(idx_hbm, out_hbm, idx_v, *bufs_and_sems):
    bufs = bufs_and_sems[:NBUF]
    sems = bufs_and_sems[NBUF:]
    cid = lax.axis_index("c")
    sid = lax.axis_index("s")
    wid = sid * NUM_CORES + cid
    cb = wid % NCB
    vlo = (wid // NCB) * VSTEP

    pltpu.sync_copy(idx_hbm.at[:, pl.ds(cb * CB, CB)], idx_v)

    lanes = lax.iota(jnp.int32, 16)
    ones = jnp.ones((LANES,), jnp.float32)
    zeros = jnp.zeros((LANES,), jnp.float32)

    for buf in bufs:
        def zero_row(r, c, buf=buf):
            for j in range(CB // LANES):
                buf[r, pl.ds(j * LANES, LANES)] = zeros
            return c

        lax.fori_loop(0, VN, zero_row, 0)

    def scatter_slab(buf, s, vals):
        for g in range(BGROUPS):
            bloc = g * LANES + lanes
            v = idx_v[s, pl.ds(g * LANES, LANES)]
            vloc = v - vlo
            mask = (vloc >= 0) & (vloc < VN)
            plsc.store_scatter(buf, [vloc, bloc], vals, mask=mask)

    def dma(r, s):
        return pltpu.make_async_copy(
            bufs[r],
            out_hbm.at[s, pl.ds(vlo, VN), pl.ds(cb * CB, CB)],
            sems[r],
        )

    def pair_body(p, c):
        for r in range(NBUF):
            s = p * NBUF + r

            @pl.when(p > 0)
            def _():
                dma(r, s - NBUF).wait()
                scatter_slab(bufs[r], s - NBUF, zeros)

            scatter_slab(bufs[r], s, ones)
            dma(r, s).start()
        return c

    lax.fori_loop(0, SEQ_LEN // NBUF, pair_body, 0)

    for r in range(NBUF):
        dma(r, SEQ_LEN - NBUF + r).wait()


def kernel(indices, table):
    del table
    out_t = _onehot_sc(indices.T)
    return jnp.transpose(out_t, (2, 0, 1))
